# baseline (speedup 1.0000x reference)
import functools

import jax
import jax.numpy as jnp
from jax import lax
from jax.experimental import pallas as pl
from jax.experimental.pallas import tpu as pltpu

N = 4096
D = 64
BM = 512
BK = 512
NI = N // BM
NK = N // BK

_F32 = jnp.float32
_BF16 = jnp.bfloat16


def _body(a0_ref, c0_ref, a1_ref, c1_ref, n_ref,
          x0_ref, x1_ref, w0_ref, w1_ref, ws_ref, wt_ref, wa0_ref, wa1_ref,
          out0_ref, out1_ref,
          h0_ref, h1_ref, hs_ref, ht_ref, accT_ref, x11_ref,
          acc00_ref, acc11_ref, acc10_ref):
    i = pl.program_id(0)
    k = pl.program_id(1)

    @pl.when((i == 0) & (k == 0))
    def _prologue():
        h0_ref[...] = jnp.dot(x0_ref[...], w0_ref[...],
                              preferred_element_type=_F32).astype(_BF16)
        h1_ref[...] = jnp.dot(x1_ref[...], w1_ref[...],
                              preferred_element_type=_F32).astype(_BF16)
        hs_ref[...] = jnp.dot(x1_ref[...], ws_ref[...],
                              preferred_element_type=_F32).astype(_BF16)
        ht_ref[...] = jnp.dot(x0_ref[...], wt_ref[...],
                              preferred_element_type=_F32).astype(_BF16)

    @pl.when(k == 0)
    def _zero_row_accs():
        acc00_ref[...] = jnp.zeros_like(acc00_ref)
        acc11_ref[...] = jnp.zeros_like(acc11_ref)
        acc10_ref[...] = jnp.zeros_like(acc10_ref)

    h0k = h0_ref[pl.ds(k * BK, BK), :]
    h1k = h1_ref[pl.ds(k * BK, BK), :]
    hsk = hs_ref[pl.ds(k * BK, BK), :]
    hti = ht_ref[pl.ds(i * BM, BM), :]

    nblk = n_ref[...].astype(_BF16)
    acc00_ref[...] += jnp.dot((a0_ref[...] * c0_ref[...]).astype(_BF16), h0k,
                              preferred_element_type=_F32)
    acc11_ref[...] += jnp.dot((a1_ref[...] * c1_ref[...]).astype(_BF16), h1k,
                              preferred_element_type=_F32)
    acc10_ref[...] += jnp.dot(nblk, hsk, preferred_element_type=_F32)
    pT = lax.dot_general(nblk, hti, (((0,), (0,)), ((), ())),
                         preferred_element_type=_F32)

    @pl.when(i == 0)
    def _init_accT():
        accT_ref[pl.ds(k * BK, BK), :] = pT

    @pl.when(i > 0)
    def _add_accT():
        accT_ref[pl.ds(k * BK, BK), :] += pT

    @pl.when(k == NK - 1)
    def _finish_row_block():
        m0 = jax.nn.relu(acc00_ref[...]) + jax.nn.relu(acc10_ref[...])
        out0_ref[pl.ds(i * BM, BM), :] = jax.nn.relu(
            jnp.dot(m0, wa0_ref[...], preferred_element_type=_F32))
        x11_ref[pl.ds(i * BM, BM), :] = jax.nn.relu(acc11_ref[...])

    @pl.when(i == NI - 1)
    def _finish_col_block():
        m1 = x11_ref[pl.ds(k * BK, BK), :] + jax.nn.relu(accT_ref[pl.ds(k * BK, BK), :])
        out1_ref[pl.ds(k * BK, BK), :] = jax.nn.relu(
            jnp.dot(m1, wa1_ref[...], preferred_element_type=_F32))


@jax.jit
def _gnn_fused(a0, c0, a1, c1, inc, x0, x1, w0, w1, ws, wt, wa0, wa1):
    big = pl.BlockSpec((BM, BK), lambda i, k: (i, k))
    full = pl.BlockSpec((N, D), lambda i, k: (0, 0))
    wspec = pl.BlockSpec((D, D), lambda i, k: (0, 0))
    out0, out1 = pl.pallas_call(
        _body,
        grid=(NI, NK),
        in_specs=[big, big, big, big, big,
                  full, full, wspec, wspec, wspec, wspec, wspec, wspec],
        out_specs=[pl.BlockSpec((N, D), lambda i, k: (0, 0)),
                   pl.BlockSpec((N, D), lambda i, k: (0, 0))],
        out_shape=[jax.ShapeDtypeStruct((N, D), _F32),
                   jax.ShapeDtypeStruct((N, D), _F32)],
        scratch_shapes=[
            pltpu.VMEM((N, D), _BF16),
            pltpu.VMEM((N, D), _BF16),
            pltpu.VMEM((N, D), _BF16),
            pltpu.VMEM((N, D), _BF16),
            pltpu.VMEM((N, D), _F32),
            pltpu.VMEM((N, D), _F32),
            pltpu.VMEM((BM, D), _F32),
            pltpu.VMEM((BM, D), _F32),
            pltpu.VMEM((BM, D), _F32),
        ],
        compiler_params=pltpu.CompilerParams(
            dimension_semantics=("arbitrary", "arbitrary")),
    )(a0, c0, a1, c1, inc, x0, x1, w0, w1, ws, wt, wa0, wa1)
    return out0, out1


def kernel(x_0, x_1, x_2, x_3, x_4, adjacency_0, adjacency_1, adjacency_2, adjacency_3, adjacency_4, incidence_0_1, incidence_0_2, incidence_0_3, incidence_0_4, incidence_1_2, incidence_1_3, incidence_1_4, incidence_2_3, incidence_2_4, incidence_3_4, cci_0_to_0, cci_1_to_1, cci_2_to_2, cci_3_to_3, cci_4_to_4, cci_0_to_1, cci_0_to_2, cci_0_to_3, cci_0_to_4, cci_1_to_2, cci_1_to_3, cci_1_to_4, cci_2_to_3, cci_2_to_4, cci_3_to_4, W_hbs0, W_hbs1, W_hbns_s, W_hbns_t, W_agg0, W_agg1):
    out0, out1 = _gnn_fused(adjacency_0, cci_0_to_0, adjacency_1, cci_1_to_1,
                            incidence_0_1, x_0, x_1,
                            W_hbs0, W_hbs1, W_hbns_s, W_hbns_t, W_agg0, W_agg1)
    return (out0, out1, x_2, x_3, x_4)


# speedup vs baseline: 1.0955x; 1.0955x over previous
import jax
import jax.numpy as jnp
from jax import lax
from jax.experimental import pallas as pl
from jax.experimental.pallas import tpu as pltpu

N = 4096
D = 64
BM = 256
NI = N // BM

_F32 = jnp.float32
_BF16 = jnp.bfloat16


def _body(a0_ref, c0_ref, a1_ref, c1_ref, n_ref,
          x0_ref, x1_ref, w0_ref, w1_ref, ws_ref, wt_ref, wa0_ref, wa1_ref,
          out0_ref, out1_ref,
          h0_ref, h1_ref, hs_ref, ht_ref, accT_ref, x11_ref):
    i = pl.program_id(0)

    @pl.when(i == 0)
    def _prologue():
        h0_ref[...] = jnp.dot(x0_ref[...], w0_ref[...],
                              preferred_element_type=_F32).astype(_BF16)
        h1_ref[...] = jnp.dot(x1_ref[...], w1_ref[...],
                              preferred_element_type=_F32).astype(_BF16)
        hs_ref[...] = jnp.dot(x1_ref[...], ws_ref[...],
                              preferred_element_type=_F32).astype(_BF16)
        ht_ref[...] = jnp.dot(x0_ref[...], wt_ref[...],
                              preferred_element_type=_F32).astype(_BF16)

    d00 = jnp.dot((a0_ref[...] * c0_ref[...]).astype(_BF16), h0_ref[...],
                  preferred_element_type=_F32)
    d11 = jnp.dot((a1_ref[...] * c1_ref[...]).astype(_BF16), h1_ref[...],
                  preferred_element_type=_F32)
    nbf = n_ref[...].astype(_BF16)
    d10 = jnp.dot(nbf, hs_ref[...], preferred_element_type=_F32)
    pT = lax.dot_general(nbf, ht_ref[pl.ds(i * BM, BM), :],
                         (((0,), (0,)), ((), ())), preferred_element_type=_F32)

    @pl.when(i == 0)
    def _init_accT():
        accT_ref[...] = pT

    @pl.when(i > 0)
    def _add_accT():
        accT_ref[...] += pT

    m0 = jax.nn.relu(d00) + jax.nn.relu(d10)
    out0_ref[pl.ds(i * BM, BM), :] = jax.nn.relu(
        jnp.dot(m0, wa0_ref[...], preferred_element_type=_F32))
    x11_ref[pl.ds(i * BM, BM), :] = jax.nn.relu(d11)

    @pl.when(i == NI - 1)
    def _epilogue():
        m1 = x11_ref[...] + jax.nn.relu(accT_ref[...])
        out1_ref[...] = jax.nn.relu(
            jnp.dot(m1, wa1_ref[...], preferred_element_type=_F32))


@jax.jit
def _gnn_fused(a0, c0, a1, c1, inc, x0, x1, w0, w1, ws, wt, wa0, wa1):
    big = pl.BlockSpec((BM, N), lambda i: (i, 0))
    full = pl.BlockSpec((N, D), lambda i: (0, 0))
    wspec = pl.BlockSpec((D, D), lambda i: (0, 0))
    out0, out1 = pl.pallas_call(
        _body,
        grid=(NI,),
        in_specs=[big, big, big, big, big,
                  full, full, wspec, wspec, wspec, wspec, wspec, wspec],
        out_specs=[pl.BlockSpec((N, D), lambda i: (0, 0)),
                   pl.BlockSpec((N, D), lambda i: (0, 0))],
        out_shape=[jax.ShapeDtypeStruct((N, D), _F32),
                   jax.ShapeDtypeStruct((N, D), _F32)],
        scratch_shapes=[
            pltpu.VMEM((N, D), _BF16),
            pltpu.VMEM((N, D), _BF16),
            pltpu.VMEM((N, D), _BF16),
            pltpu.VMEM((N, D), _BF16),
            pltpu.VMEM((N, D), _F32),
            pltpu.VMEM((N, D), _F32),
        ],
        compiler_params=pltpu.CompilerParams(
            dimension_semantics=("arbitrary",)),
    )(a0, c0, a1, c1, inc, x0, x1, w0, w1, ws, wt, wa0, wa1)
    return out0, out1


def kernel(x_0, x_1, x_2, x_3, x_4, adjacency_0, adjacency_1, adjacency_2, adjacency_3, adjacency_4, incidence_0_1, incidence_0_2, incidence_0_3, incidence_0_4, incidence_1_2, incidence_1_3, incidence_1_4, incidence_2_3, incidence_2_4, incidence_3_4, cci_0_to_0, cci_1_to_1, cci_2_to_2, cci_3_to_3, cci_4_to_4, cci_0_to_1, cci_0_to_2, cci_0_to_3, cci_0_to_4, cci_1_to_2, cci_1_to_3, cci_1_to_4, cci_2_to_3, cci_2_to_4, cci_3_to_4, W_hbs0, W_hbs1, W_hbns_s, W_hbns_t, W_agg0, W_agg1):
    out0, out1 = _gnn_fused(adjacency_0, cci_0_to_0, adjacency_1, cci_1_to_1,
                            incidence_0_1, x_0, x_1,
                            W_hbs0, W_hbs1, W_hbns_s, W_hbns_t, W_agg0, W_agg1)
    return (out0, out1, x_2, x_3, x_4)
